# baseline (speedup 1.0000x reference)
import dataclasses
import functools
import jax
import jax.numpy as jnp
from jax import lax
from jax.experimental import pallas as pl
from jax.experimental.pallas import tpu as pltpu
from jax.experimental.pallas import tpu_sc as plsc

NC, NS, LANES = 2, 16, 16
NW = NC * NS
CHK = 128
RC = 400

_MESH = dict(core_axis_name="c", subcore_axis_name="s", num_cores=NC,
             num_subcores=NS)

_SC_PARAMS = dataclasses.replace(pltpu.CompilerParams(),
                                 needs_layout_passes=False,
                                 use_tc_tiling_on_sc=False)


def _sc_deg(dst3, ew3, zero16, n_nodes, tpc):
    @functools.partial(
        pl.kernel,
        out_type=jax.ShapeDtypeStruct((NC, n_nodes, LANES), jnp.float32),
        mesh=plsc.VectorSubcoreMesh(**_MESH),
        compiler_params=_SC_PARAMS,
        scratch_types=[
            pltpu.VMEM_SHARED((n_nodes, LANES), jnp.float32),
            pltpu.VMEM((tpc, CHK), jnp.int32),
            pltpu.VMEM((tpc, CHK), jnp.float32),
            pltpu.VMEM((CHK, LANES), jnp.float32),
            pltpu.VMEM((CHK, LANES), jnp.float32),
            pltpu.SemaphoreType.DMA,
            pltpu.SemaphoreType.DMA,
            pltpu.SemaphoreType.DMA,
        ],
    )
    def deg_kernel(dst3_hbm, ew3_hbm, zero_hbm, degp_hbm, shared, didx2, ewf,
                   rows0, rows1, sem, s0, s1):
        cid = lax.axis_index("c")
        sid = lax.axis_index("s")
        wid = sid * NC + cid
        n_rchunks = n_nodes // RC
        rows = (rows0, rows1)
        ssems = (s0, s1)

        cd = pltpu.async_copy(dst3_hbm.at[wid], didx2, s0)
        ce = pltpu.async_copy(ew3_hbm.at[wid], ewf, s1)

        @pl.loop(sid, n_rchunks, step=NS)
        def _(rc):
            r0 = pl.multiple_of(rc * RC, RC)
            pltpu.async_copy(zero_hbm.at[pl.ds(r0, RC)],
                             shared.at[pl.ds(r0, RC)], sem).wait()

        pltpu.async_copy(zero_hbm.at[pl.ds(0, CHK)], rows0, sem).wait()
        pltpu.async_copy(zero_hbm.at[pl.ds(0, CHK)], rows1, sem).wait()
        cd.wait()
        ce.wait()
        plsc.subcore_barrier()

        iota16 = lax.broadcasted_iota(jnp.int32, (LANES,), 0)
        lane0 = jnp.zeros((LANES,), jnp.int32)

        @pl.loop(0, tpc // 2)
        def _(k):
            for b in range(2):
                ci = k * 2 + b

                def wait_prev(b=b, ci=ci):
                    pltpu.make_async_copy(
                        rows[b], shared.at[didx2.at[ci]], ssems[b]).wait()

                pl.when(k >= 1)(wait_prev)

                for grp in range(CHK // LANES):
                    ew16 = ewf[ci, pl.ds(grp * LANES, LANES)]
                    plsc.store_scatter(rows[b], [iota16 + grp * LANES, lane0],
                                       ew16)

                pltpu.async_copy(rows[b], shared.at[didx2.at[ci]], ssems[b],
                                 add=True)

        pltpu.make_async_copy(rows0, shared.at[didx2.at[tpc - 2]], s0).wait()
        pltpu.make_async_copy(rows1, shared.at[didx2.at[tpc - 1]], s1).wait()
        plsc.subcore_barrier()

        @pl.loop(sid, n_rchunks, step=NS)
        def _(rc):
            r0 = pl.multiple_of(rc * RC, RC)
            pltpu.async_copy(shared.at[pl.ds(r0, RC)],
                             degp_hbm.at[cid, pl.ds(r0, RC)], sem).wait()

    return deg_kernel(dst3, ew3, zero16)


def _sc_msgs(src3, dst3, ew3, g, zero_d, n_nodes, d_out, tpc):
    n_vec = d_out // LANES

    @functools.partial(
        pl.kernel,
        out_type=jax.ShapeDtypeStruct((NC, n_nodes, d_out), jnp.float32),
        mesh=plsc.VectorSubcoreMesh(**_MESH),
        compiler_params=_SC_PARAMS,
        scratch_types=[
            pltpu.VMEM_SHARED((n_nodes, d_out), jnp.float32),
            pltpu.VMEM((tpc, CHK), jnp.int32),
            pltpu.VMEM((CHK,), jnp.int32),
            pltpu.VMEM((CHK,), jnp.int32),
            pltpu.VMEM((CHK,), jnp.float32),
            pltpu.VMEM((CHK,), jnp.float32),
            pltpu.VMEM((CHK, d_out), jnp.float32),
            pltpu.VMEM((CHK, d_out), jnp.float32),
            pltpu.SemaphoreType.DMA,
            pltpu.SemaphoreType.DMA,
            pltpu.SemaphoreType.DMA,
            pltpu.SemaphoreType.DMA,
            pltpu.SemaphoreType.DMA,
            pltpu.SemaphoreType.DMA,
            pltpu.SemaphoreType.DMA,
        ],
    )
    def msg_kernel(src3_hbm, dst3_hbm, ew3_hbm, g_hbm, zero_hbm, accp_hbm,
                   acc_sh, didx2, sidx0, sidx1, ewb0, ewb1, rows0, rows1,
                   zsem, g0, g1, t0, t1, l0, l1):
        cid = lax.axis_index("c")
        sid = lax.axis_index("s")
        wid = sid * NC + cid
        n_rchunks = n_nodes // RC
        rows = (rows0, rows1)
        sidx = (sidx0, sidx1)
        ewb = (ewb0, ewb1)
        gs = (g0, g1)
        ts = (t0, t1)
        ls = (l0, l1)

        def issue_loads(ci, b):
            pltpu.async_copy(src3_hbm.at[wid, ci], sidx[b], ls[b])
            pltpu.async_copy(ew3_hbm.at[wid, ci], ewb[b], ls[b])

        def wait_loads(b):
            pltpu.make_async_copy(src3_hbm.at[wid, 0], sidx[b], ls[b]).wait()
            pltpu.make_async_copy(ew3_hbm.at[wid, 0], ewb[b], ls[b]).wait()

        cd = pltpu.async_copy(dst3_hbm.at[wid], didx2, g1)

        @pl.loop(sid, n_rchunks, step=NS)
        def _(rc):
            r0 = pl.multiple_of(rc * RC, RC)
            pltpu.async_copy(zero_hbm.at[pl.ds(r0, RC)],
                             acc_sh.at[pl.ds(r0, RC)], zsem).wait()

        cd.wait()
        issue_loads(0, 0)
        issue_loads(1, 1)
        wait_loads(0)
        pltpu.async_copy(g_hbm.at[sidx0], rows0, g0)
        plsc.subcore_barrier()

        @pl.loop(0, tpc // 2)
        def _(k):
            for b in range(2):
                ci = k * 2 + b

                pltpu.make_async_copy(g_hbm.at[sidx[b]], rows[b],
                                      gs[b]).wait()

                def wait_scatter(b=b, ci=ci):
                    pltpu.make_async_copy(
                        rows[1 - b], acc_sh.at[didx2.at[ci]], ts[1 - b]).wait()

                if b == 0:
                    pl.when(k >= 1)(wait_scatter)
                else:
                    wait_scatter()

                def wait_idx(b=b):
                    wait_loads(1 - b)

                def issue_gather(b=b, ci=ci):
                    pltpu.async_copy(g_hbm.at[sidx[1 - b]], rows[1 - b],
                                     gs[1 - b])

                if b == 0:
                    wait_idx()
                    issue_gather()
                else:
                    def prep_next(b=b, ci=ci):
                        wait_idx(b)
                        issue_gather(b, ci)

                    pl.when(k + 1 < tpc // 2)(prep_next)

                @pl.loop(0, CHK)
                def _(e, b=b):
                    sp = plsc.load_gather(ewb[b],
                                          [lax.broadcast(e, (LANES,))])
                    for j in range(n_vec):
                        sl = pl.ds(j * LANES, LANES)
                        rows[b][e, sl] = rows[b][e, sl] * sp

                pltpu.async_copy(rows[b], acc_sh.at[didx2.at[ci]], ts[b],
                                 add=True)

                def issue_next_loads(b=b, ci=ci):
                    issue_loads(ci + 2, b)

                pl.when(ci + 2 < tpc)(issue_next_loads)

        pltpu.make_async_copy(rows1, acc_sh.at[didx2.at[tpc - 1]], t1).wait()
        plsc.subcore_barrier()

        @pl.loop(sid, n_rchunks, step=NS)
        def _(rc):
            r0 = pl.multiple_of(rc * RC, RC)
            pltpu.async_copy(acc_sh.at[pl.ds(r0, RC)],
                             accp_hbm.at[cid, pl.ds(r0, RC)], zsem).wait()

    return msg_kernel(src3, dst3, ew3, g, zero_d)


def _tc_g(x, W, degp, block_n):
    n, d_in = x.shape
    d_out = W.shape[1]

    def body(x_ref, w_ref, degp_ref, g_ref):
        h = jnp.dot(x_ref[...], w_ref[...], preferred_element_type=jnp.float32)
        deg = degp_ref[0] + degp_ref[1] + 1.0
        dis = jnp.where(deg > 0, lax.rsqrt(jnp.maximum(deg, 1e-38)), 0.0)
        g_ref[...] = h * dis[:, 0:1]

    return pl.pallas_call(
        body,
        grid=(n // block_n,),
        in_specs=[
            pl.BlockSpec((block_n, d_in), lambda i: (i, 0)),
            pl.BlockSpec((d_in, d_out), lambda i: (0, 0)),
            pl.BlockSpec((NC, block_n, LANES), lambda i: (0, i, 0)),
        ],
        out_specs=pl.BlockSpec((block_n, d_out), lambda i: (i, 0)),
        out_shape=jax.ShapeDtypeStruct((n, d_out), jnp.float32),
    )(x, W, degp)


def _tc_out(accp, g, degp, b2d, block_n):
    n, d_out = g.shape

    def body(accp_ref, g_ref, degp_ref, b_ref, o_ref):
        s = accp_ref[0] + accp_ref[1] + g_ref[...]
        deg = degp_ref[0] + degp_ref[1] + 1.0
        dis = jnp.where(deg > 0, lax.rsqrt(jnp.maximum(deg, 1e-38)), 0.0)
        z = s * dis[:, 0:1] + b_ref[...]
        m = jnp.max(z, axis=-1, keepdims=True)
        lse = m + jnp.log(jnp.sum(jnp.exp(z - m), axis=-1, keepdims=True))
        o_ref[...] = z - lse

    return pl.pallas_call(
        body,
        grid=(n // block_n,),
        in_specs=[
            pl.BlockSpec((NC, block_n, d_out), lambda i: (0, i, 0)),
            pl.BlockSpec((block_n, d_out), lambda i: (i, 0)),
            pl.BlockSpec((NC, block_n, LANES), lambda i: (0, i, 0)),
            pl.BlockSpec((1, d_out), lambda i: (0, 0)),
        ],
        out_specs=pl.BlockSpec((block_n, d_out), lambda i: (i, 0)),
        out_shape=jax.ShapeDtypeStruct((n, d_out), jnp.float32),
    )(accp, g, degp, b2d)


@jax.jit
def kernel(x, edge_index, edge_weight, W, b):
    n_nodes, _ = x.shape
    d_out = W.shape[1]
    n_edges = edge_index.shape[1]
    src = edge_index[0].astype(jnp.int32)
    dst = edge_index[1].astype(jnp.int32)
    ew = edge_weight.astype(jnp.float32)

    grp = NW * CHK
    tpc = -(-n_edges // grp)
    tpc += tpc % 2
    pad = tpc * grp - n_edges
    src = jnp.concatenate([src, jnp.zeros((pad,), jnp.int32)])
    dst = jnp.concatenate([dst, jnp.zeros((pad,), jnp.int32)])
    ew = jnp.concatenate([ew, jnp.zeros((pad,), jnp.float32)])
    src3 = src.reshape(NW, tpc, CHK)
    dst3 = dst.reshape(NW, tpc, CHK)
    ew3 = ew.reshape(NW, tpc, CHK)

    zero16 = jnp.zeros((n_nodes, LANES), jnp.float32)
    zero_d = jnp.zeros((n_nodes, d_out), jnp.float32)

    degp = _sc_deg(dst3, ew3, zero16, n_nodes, tpc)
    g = _tc_g(x, W, degp, block_n=2000)
    accp = _sc_msgs(src3, dst3, ew3, g, zero_d, n_nodes, d_out, tpc)
    return _tc_out(accp, g, degp, jnp.reshape(b, (1, d_out)), block_n=2000)


# speedup vs baseline: 2.4251x; 2.4251x over previous
import dataclasses
import functools
import jax
import jax.numpy as jnp
from jax import lax
from jax.experimental import pallas as pl
from jax.experimental.pallas import tpu as pltpu
from jax.experimental.pallas import tpu_sc as plsc

NC, NS, LANES = 2, 16, 16
NW = NC * NS
CHK = 128
RC = 400

_MESH = dict(core_axis_name="c", subcore_axis_name="s", num_cores=NC,
             num_subcores=NS)

_SC_PARAMS = dataclasses.replace(pltpu.CompilerParams(),
                                 needs_layout_passes=False,
                                 use_tc_tiling_on_sc=False)


def _sc_deg(dst3, ew3, zero16, n_nodes, tpc):
    @functools.partial(
        pl.kernel,
        out_type=jax.ShapeDtypeStruct((NC, n_nodes, LANES), jnp.float32),
        mesh=plsc.VectorSubcoreMesh(**_MESH),
        compiler_params=_SC_PARAMS,
        scratch_types=[
            pltpu.VMEM_SHARED((n_nodes, LANES), jnp.float32),
            pltpu.VMEM((tpc, CHK), jnp.int32),
            pltpu.VMEM((tpc, CHK), jnp.float32),
            pltpu.VMEM((CHK, LANES), jnp.float32),
            pltpu.VMEM((CHK, LANES), jnp.float32),
            pltpu.SemaphoreType.DMA,
            pltpu.SemaphoreType.DMA,
            pltpu.SemaphoreType.DMA,
        ],
    )
    def deg_kernel(dst3_hbm, ew3_hbm, zero_hbm, degp_hbm, shared, didx2, ewf,
                   rows0, rows1, sem, s0, s1):
        cid = lax.axis_index("c")
        sid = lax.axis_index("s")
        wid = sid * NC + cid
        n_rchunks = n_nodes // RC
        rows = (rows0, rows1)
        ssems = (s0, s1)

        cd = pltpu.async_copy(dst3_hbm.at[wid], didx2, s0)
        ce = pltpu.async_copy(ew3_hbm.at[wid], ewf, s1)

        @pl.loop(sid, n_rchunks, step=NS)
        def _(rc):
            r0 = pl.multiple_of(rc * RC, RC)
            pltpu.async_copy(zero_hbm.at[pl.ds(r0, RC)],
                             shared.at[pl.ds(r0, RC)], sem).wait()

        pltpu.async_copy(zero_hbm.at[pl.ds(0, CHK)], rows0, sem).wait()
        pltpu.async_copy(zero_hbm.at[pl.ds(0, CHK)], rows1, sem).wait()
        cd.wait()
        ce.wait()
        plsc.subcore_barrier()

        iota16 = lax.broadcasted_iota(jnp.int32, (LANES,), 0)
        lane0 = jnp.zeros((LANES,), jnp.int32)

        @pl.loop(0, tpc // 2)
        def _(k):
            for b in range(2):
                ci = k * 2 + b

                def wait_prev(b=b, ci=ci):
                    pltpu.make_async_copy(
                        rows[b], shared.at[didx2.at[ci]], ssems[b]).wait()

                pl.when(k >= 1)(wait_prev)

                for grp in range(CHK // LANES):
                    ew16 = ewf[ci, pl.ds(grp * LANES, LANES)]
                    plsc.store_scatter(rows[b], [iota16 + grp * LANES, lane0],
                                       ew16)

                pltpu.async_copy(rows[b], shared.at[didx2.at[ci]], ssems[b],
                                 add=True)

        pltpu.make_async_copy(rows0, shared.at[didx2.at[tpc - 2]], s0).wait()
        pltpu.make_async_copy(rows1, shared.at[didx2.at[tpc - 1]], s1).wait()
        plsc.subcore_barrier()

        @pl.loop(sid, n_rchunks, step=NS)
        def _(rc):
            r0 = pl.multiple_of(rc * RC, RC)
            pltpu.async_copy(shared.at[pl.ds(r0, RC)],
                             degp_hbm.at[cid, pl.ds(r0, RC)], sem).wait()

    return deg_kernel(dst3, ew3, zero16)


def _sc_msgs(src3, dst3, ew3, g, zero_d, n_nodes, d_out, tpc):
    n_vec = d_out // LANES

    @functools.partial(
        pl.kernel,
        out_type=jax.ShapeDtypeStruct((NC, n_nodes, d_out), jnp.float32),
        mesh=plsc.VectorSubcoreMesh(**_MESH),
        compiler_params=_SC_PARAMS,
        scratch_types=[
            pltpu.VMEM_SHARED((n_nodes, d_out), jnp.float32),
            pltpu.VMEM((tpc, CHK), jnp.int32),
            pltpu.VMEM((CHK,), jnp.int32),
            pltpu.VMEM((CHK,), jnp.int32),
            pltpu.VMEM((CHK,), jnp.float32),
            pltpu.VMEM((CHK,), jnp.float32),
            pltpu.VMEM((CHK, d_out), jnp.float32),
            pltpu.VMEM((CHK, d_out), jnp.float32),
            pltpu.SemaphoreType.DMA,
            pltpu.SemaphoreType.DMA,
            pltpu.SemaphoreType.DMA,
            pltpu.SemaphoreType.DMA,
            pltpu.SemaphoreType.DMA,
            pltpu.SemaphoreType.DMA,
            pltpu.SemaphoreType.DMA,
        ],
    )
    def msg_kernel(src3_hbm, dst3_hbm, ew3_hbm, g_hbm, zero_hbm, accp_hbm,
                   acc_sh, didx2, sidx0, sidx1, ewb0, ewb1, rows0, rows1,
                   zsem, g0, g1, t0, t1, l0, l1):
        cid = lax.axis_index("c")
        sid = lax.axis_index("s")
        wid = sid * NC + cid
        n_rchunks = n_nodes // RC
        rows = (rows0, rows1)
        sidx = (sidx0, sidx1)
        ewb = (ewb0, ewb1)
        gs = (g0, g1)
        ts = (t0, t1)
        ls = (l0, l1)

        def issue_loads(ci, b):
            pltpu.async_copy(src3_hbm.at[wid, ci], sidx[b], ls[b])
            pltpu.async_copy(ew3_hbm.at[wid, ci], ewb[b], ls[b])

        def wait_loads(b):
            pltpu.make_async_copy(src3_hbm.at[wid, 0], sidx[b], ls[b]).wait()
            pltpu.make_async_copy(ew3_hbm.at[wid, 0], ewb[b], ls[b]).wait()

        cd = pltpu.async_copy(dst3_hbm.at[wid], didx2, g1)

        @pl.loop(sid, n_rchunks, step=NS)
        def _(rc):
            r0 = pl.multiple_of(rc * RC, RC)
            pltpu.async_copy(zero_hbm.at[pl.ds(r0, RC)],
                             acc_sh.at[pl.ds(r0, RC)], zsem).wait()

        cd.wait()
        issue_loads(0, 0)
        issue_loads(1, 1)
        wait_loads(0)
        pltpu.async_copy(g_hbm.at[sidx0], rows0, g0)
        plsc.subcore_barrier()

        @pl.loop(0, tpc // 2)
        def _(k):
            for b in range(2):
                ci = k * 2 + b

                pltpu.make_async_copy(g_hbm.at[sidx[b]], rows[b],
                                      gs[b]).wait()

                def wait_scatter(b=b, ci=ci):
                    pltpu.make_async_copy(
                        rows[1 - b], acc_sh.at[didx2.at[ci]], ts[1 - b]).wait()

                if b == 0:
                    pl.when(k >= 1)(wait_scatter)
                else:
                    wait_scatter()

                def wait_idx(b=b):
                    wait_loads(1 - b)

                def issue_gather(b=b, ci=ci):
                    pltpu.async_copy(g_hbm.at[sidx[1 - b]], rows[1 - b],
                                     gs[1 - b])

                if b == 0:
                    wait_idx()
                    issue_gather()
                else:
                    def prep_next(b=b, ci=ci):
                        wait_idx(b)
                        issue_gather(b, ci)

                    pl.when(k + 1 < tpc // 2)(prep_next)

                @pl.loop(0, CHK)
                def _(e, b=b):
                    sp = plsc.load_gather(ewb[b],
                                          [lax.broadcast(e, (LANES,))])
                    for j in range(n_vec):
                        sl = pl.ds(j * LANES, LANES)
                        rows[b][e, sl] = rows[b][e, sl] * sp

                pltpu.async_copy(rows[b], acc_sh.at[didx2.at[ci]], ts[b],
                                 add=True)

                def issue_next_loads(b=b, ci=ci):
                    issue_loads(ci + 2, b)

                pl.when(ci + 2 < tpc)(issue_next_loads)

        pltpu.make_async_copy(rows1, acc_sh.at[didx2.at[tpc - 1]], t1).wait()
        plsc.subcore_barrier()

        @pl.loop(sid, n_rchunks, step=NS)
        def _(rc):
            r0 = pl.multiple_of(rc * RC, RC)
            pltpu.async_copy(acc_sh.at[pl.ds(r0, RC)],
                             accp_hbm.at[cid, pl.ds(r0, RC)], zsem).wait()

    return msg_kernel(src3, dst3, ew3, g, zero_d)


def _tc_g(x, W, degp, block_n):
    n, d_in = x.shape
    d_out = W.shape[1]

    def body(x_ref, w_ref, degp_ref, g_ref):
        h = jnp.dot(x_ref[...], w_ref[...], preferred_element_type=jnp.float32)
        deg = degp_ref[0] + degp_ref[1] + 1.0
        dis = jnp.where(deg > 0, lax.rsqrt(jnp.maximum(deg, 1e-38)), 0.0)
        g_ref[...] = h * dis[:, 0:1]

    return pl.pallas_call(
        body,
        grid=(n // block_n,),
        in_specs=[
            pl.BlockSpec((block_n, d_in), lambda i: (i, 0)),
            pl.BlockSpec((d_in, d_out), lambda i: (0, 0)),
            pl.BlockSpec((NC, block_n, LANES), lambda i: (0, i, 0)),
        ],
        out_specs=pl.BlockSpec((block_n, d_out), lambda i: (i, 0)),
        out_shape=jax.ShapeDtypeStruct((n, d_out), jnp.float32),
    )(x, W, degp)


def _tc_out(accp, g, degp, b2d, block_n):
    n, d_out = g.shape

    def body(accp_ref, g_ref, degp_ref, b_ref, o_ref):
        s = accp_ref[0] + accp_ref[1] + g_ref[...]
        deg = degp_ref[0] + degp_ref[1] + 1.0
        dis = jnp.where(deg > 0, lax.rsqrt(jnp.maximum(deg, 1e-38)), 0.0)
        z = s * dis[:, 0:1] + b_ref[...]
        m = jnp.max(z, axis=-1, keepdims=True)
        lse = m + jnp.log(jnp.sum(jnp.exp(z - m), axis=-1, keepdims=True))
        o_ref[...] = z - lse

    return pl.pallas_call(
        body,
        grid=(n // block_n,),
        in_specs=[
            pl.BlockSpec((NC, block_n, d_out), lambda i: (0, i, 0)),
            pl.BlockSpec((block_n, d_out), lambda i: (i, 0)),
            pl.BlockSpec((NC, block_n, LANES), lambda i: (0, i, 0)),
            pl.BlockSpec((1, d_out), lambda i: (0, 0)),
        ],
        out_specs=pl.BlockSpec((block_n, d_out), lambda i: (i, 0)),
        out_shape=jax.ShapeDtypeStruct((n, d_out), jnp.float32),
    )(accp, g, degp, b2d)


@jax.jit
def kernel(x, edge_index, edge_weight, W, b):
    n_nodes, _ = x.shape
    d_out = W.shape[1]
    n_edges = edge_index.shape[1]
    src = edge_index[0].astype(jnp.int32)
    dst = edge_index[1].astype(jnp.int32)
    ew = edge_weight.astype(jnp.float32)

    grp = NW * CHK
    tpc = -(-n_edges // grp)
    tpc += tpc % 2
    pad = tpc * grp - n_edges
    spread = jnp.arange(pad, dtype=jnp.int32) % n_nodes
    src = jnp.concatenate([src, spread])
    dst = jnp.concatenate([dst, spread])
    ew = jnp.concatenate([ew, jnp.zeros((pad,), jnp.float32)])
    src3 = src.reshape(NW, tpc, CHK)
    dst3 = dst.reshape(NW, tpc, CHK)
    ew3 = ew.reshape(NW, tpc, CHK)

    zero16 = jnp.zeros((n_nodes, LANES), jnp.float32)
    zero_d = jnp.zeros((n_nodes, d_out), jnp.float32)

    degp = _sc_deg(dst3, ew3, zero16, n_nodes, tpc)
    g = _tc_g(x, W, degp, block_n=2000)
    accp = _sc_msgs(src3, dst3, ew3, g, zero_d, n_nodes, d_out, tpc)
    return _tc_out(accp, g, degp, jnp.reshape(b, (1, d_out)), block_n=2000)


# speedup vs baseline: 2.8300x; 1.1669x over previous
import dataclasses
import functools
import jax
import jax.numpy as jnp
from jax import lax
from jax.experimental import pallas as pl
from jax.experimental.pallas import tpu as pltpu
from jax.experimental.pallas import tpu_sc as plsc

NC, NS, LANES = 2, 16, 16
NW = NC * NS
CHK = 128
RC = 400

_MESH = dict(core_axis_name="c", subcore_axis_name="s", num_cores=NC,
             num_subcores=NS)

_SC_PARAMS = dataclasses.replace(pltpu.CompilerParams(),
                                 needs_layout_passes=False,
                                 use_tc_tiling_on_sc=False)


def _sc_deg(dst3, ew3, zero16, n_nodes, tpc):
    @functools.partial(
        pl.kernel,
        out_type=jax.ShapeDtypeStruct((NC, n_nodes, LANES), jnp.float32),
        mesh=plsc.VectorSubcoreMesh(**_MESH),
        compiler_params=_SC_PARAMS,
        scratch_types=[
            pltpu.VMEM_SHARED((n_nodes, LANES), jnp.float32),
            pltpu.VMEM((tpc, CHK), jnp.int32),
            pltpu.VMEM((tpc, CHK), jnp.float32),
            pltpu.VMEM((CHK, LANES), jnp.float32),
            pltpu.VMEM((CHK, LANES), jnp.float32),
            pltpu.SemaphoreType.DMA,
            pltpu.SemaphoreType.DMA,
            pltpu.SemaphoreType.DMA,
        ],
    )
    def deg_kernel(dst3_hbm, ew3_hbm, zero_hbm, degp_hbm, shared, didx2, ewf,
                   rows0, rows1, sem, s0, s1):
        cid = lax.axis_index("c")
        sid = lax.axis_index("s")
        wid = sid * NC + cid
        n_rchunks = n_nodes // RC
        rows = (rows0, rows1)
        ssems = (s0, s1)

        cd = pltpu.async_copy(dst3_hbm.at[wid], didx2, s0)
        ce = pltpu.async_copy(ew3_hbm.at[wid], ewf, s1)

        @pl.loop(sid, n_rchunks, step=NS)
        def _(rc):
            r0 = pl.multiple_of(rc * RC, RC)
            pltpu.async_copy(zero_hbm.at[pl.ds(r0, RC)],
                             shared.at[pl.ds(r0, RC)], sem).wait()

        pltpu.async_copy(zero_hbm.at[pl.ds(0, CHK)], rows0, sem).wait()
        pltpu.async_copy(zero_hbm.at[pl.ds(0, CHK)], rows1, sem).wait()
        cd.wait()
        ce.wait()
        plsc.subcore_barrier()

        iota16 = lax.broadcasted_iota(jnp.int32, (LANES,), 0)
        lane0 = jnp.zeros((LANES,), jnp.int32)

        @pl.loop(0, tpc // 2)
        def _(k):
            for b in range(2):
                ci = k * 2 + b

                def wait_prev(b=b, ci=ci):
                    pltpu.make_async_copy(
                        rows[b], shared.at[didx2.at[ci]], ssems[b]).wait()

                pl.when(k >= 1)(wait_prev)

                for grp in range(CHK // LANES):
                    ew16 = ewf[ci, pl.ds(grp * LANES, LANES)]
                    plsc.store_scatter(rows[b], [iota16 + grp * LANES, lane0],
                                       ew16)

                pltpu.async_copy(rows[b], shared.at[didx2.at[ci]], ssems[b],
                                 add=True)

        pltpu.make_async_copy(rows0, shared.at[didx2.at[tpc - 2]], s0).wait()
        pltpu.make_async_copy(rows1, shared.at[didx2.at[tpc - 1]], s1).wait()
        plsc.subcore_barrier()

        @pl.loop(sid, n_rchunks, step=NS)
        def _(rc):
            r0 = pl.multiple_of(rc * RC, RC)
            pltpu.async_copy(shared.at[pl.ds(r0, RC)],
                             degp_hbm.at[cid, pl.ds(r0, RC)], sem).wait()

    return deg_kernel(dst3, ew3, zero16)


def _sc_msgs(src3, dst3, ew3, g, zero_d, n_nodes, d_out, tpc):
    n_vec = d_out // LANES

    @functools.partial(
        pl.kernel,
        out_type=jax.ShapeDtypeStruct((NC, n_nodes, d_out), jnp.float32),
        mesh=plsc.VectorSubcoreMesh(**_MESH),
        compiler_params=_SC_PARAMS,
        scratch_types=[
            pltpu.VMEM_SHARED((n_nodes, d_out), jnp.float32),
            pltpu.VMEM((tpc, CHK), jnp.int32),
            pltpu.VMEM((CHK,), jnp.int32),
            pltpu.VMEM((CHK,), jnp.int32),
            pltpu.VMEM((CHK,), jnp.float32),
            pltpu.VMEM((CHK,), jnp.float32),
            pltpu.VMEM((CHK, d_out), jnp.float32),
            pltpu.VMEM((CHK, d_out), jnp.float32),
            pltpu.SemaphoreType.DMA,
            pltpu.SemaphoreType.DMA,
            pltpu.SemaphoreType.DMA,
            pltpu.SemaphoreType.DMA,
            pltpu.SemaphoreType.DMA,
            pltpu.SemaphoreType.DMA,
            pltpu.SemaphoreType.DMA,
        ],
    )
    def msg_kernel(src3_hbm, dst3_hbm, ew3_hbm, g_hbm, zero_hbm, accp_hbm,
                   acc_sh, didx2, sidx0, sidx1, ewb0, ewb1, rows0, rows1,
                   zsem, g0, g1, t0, t1, l0, l1):
        cid = lax.axis_index("c")
        sid = lax.axis_index("s")
        wid = sid * NC + cid
        n_rchunks = n_nodes // RC
        rows = (rows0, rows1)
        sidx = (sidx0, sidx1)
        ewb = (ewb0, ewb1)
        gs = (g0, g1)
        ts = (t0, t1)
        ls = (l0, l1)

        def issue_loads(ci, b):
            pltpu.async_copy(src3_hbm.at[wid, ci], sidx[b], ls[b])
            pltpu.async_copy(ew3_hbm.at[wid, ci], ewb[b], ls[b])

        def wait_loads(b):
            pltpu.make_async_copy(src3_hbm.at[wid, 0], sidx[b], ls[b]).wait()
            pltpu.make_async_copy(ew3_hbm.at[wid, 0], ewb[b], ls[b]).wait()

        cd = pltpu.async_copy(dst3_hbm.at[wid], didx2, g1)

        @pl.loop(sid, n_rchunks, step=NS)
        def _(rc):
            r0 = pl.multiple_of(rc * RC, RC)
            pltpu.async_copy(zero_hbm.at[pl.ds(r0, RC)],
                             acc_sh.at[pl.ds(r0, RC)], zsem).wait()

        cd.wait()
        issue_loads(0, 0)
        issue_loads(1, 1)
        wait_loads(0)
        pltpu.async_copy(g_hbm.at[sidx0], rows0, g0)
        plsc.subcore_barrier()

        @pl.loop(0, tpc // 2)
        def _(k):
            for b in range(2):
                ci = k * 2 + b

                pltpu.make_async_copy(g_hbm.at[sidx[b]], rows[b],
                                      gs[b]).wait()

                def wait_scatter(b=b, ci=ci):
                    pltpu.make_async_copy(
                        rows[1 - b], acc_sh.at[didx2.at[ci]], ts[1 - b]).wait()

                if b == 0:
                    pl.when(k >= 1)(wait_scatter)
                else:
                    wait_scatter()

                def wait_idx(b=b):
                    wait_loads(1 - b)

                def issue_gather(b=b, ci=ci):
                    pltpu.async_copy(g_hbm.at[sidx[1 - b]], rows[1 - b],
                                     gs[1 - b])

                if b == 0:
                    wait_idx()
                    issue_gather()
                else:
                    def prep_next(b=b, ci=ci):
                        wait_idx(b)
                        issue_gather(b, ci)

                    pl.when(k + 1 < tpc // 2)(prep_next)

                @plsc.parallel_loop(0, CHK, unroll=4)
                def _(e, b=b):
                    sp = plsc.load_gather(ewb[b],
                                          [lax.broadcast(e, (LANES,))])
                    for j in range(n_vec):
                        sl = pl.ds(j * LANES, LANES)
                        rows[b][e, sl] = rows[b][e, sl] * sp

                pltpu.async_copy(rows[b], acc_sh.at[didx2.at[ci]], ts[b],
                                 add=True)

                def issue_next_loads(b=b, ci=ci):
                    issue_loads(ci + 2, b)

                pl.when(ci + 2 < tpc)(issue_next_loads)

        pltpu.make_async_copy(rows1, acc_sh.at[didx2.at[tpc - 1]], t1).wait()
        plsc.subcore_barrier()

        @pl.loop(sid, n_rchunks, step=NS)
        def _(rc):
            r0 = pl.multiple_of(rc * RC, RC)
            pltpu.async_copy(acc_sh.at[pl.ds(r0, RC)],
                             accp_hbm.at[cid, pl.ds(r0, RC)], zsem).wait()

    return msg_kernel(src3, dst3, ew3, g, zero_d)


def _tc_g(x, W, degp, block_n):
    n, d_in = x.shape
    d_out = W.shape[1]

    def body(x_ref, w_ref, degp_ref, g_ref):
        h = jnp.dot(x_ref[...], w_ref[...], preferred_element_type=jnp.float32)
        deg = degp_ref[0] + degp_ref[1] + 1.0
        dis = jnp.where(deg > 0, lax.rsqrt(jnp.maximum(deg, 1e-38)), 0.0)
        g_ref[...] = h * dis[:, 0:1]

    return pl.pallas_call(
        body,
        grid=(n // block_n,),
        in_specs=[
            pl.BlockSpec((block_n, d_in), lambda i: (i, 0)),
            pl.BlockSpec((d_in, d_out), lambda i: (0, 0)),
            pl.BlockSpec((NC, block_n, LANES), lambda i: (0, i, 0)),
        ],
        out_specs=pl.BlockSpec((block_n, d_out), lambda i: (i, 0)),
        out_shape=jax.ShapeDtypeStruct((n, d_out), jnp.float32),
    )(x, W, degp)


def _tc_out(accp, g, degp, b2d, block_n):
    n, d_out = g.shape

    def body(accp_ref, g_ref, degp_ref, b_ref, o_ref):
        s = accp_ref[0] + accp_ref[1] + g_ref[...]
        deg = degp_ref[0] + degp_ref[1] + 1.0
        dis = jnp.where(deg > 0, lax.rsqrt(jnp.maximum(deg, 1e-38)), 0.0)
        z = s * dis[:, 0:1] + b_ref[...]
        m = jnp.max(z, axis=-1, keepdims=True)
        lse = m + jnp.log(jnp.sum(jnp.exp(z - m), axis=-1, keepdims=True))
        o_ref[...] = z - lse

    return pl.pallas_call(
        body,
        grid=(n // block_n,),
        in_specs=[
            pl.BlockSpec((NC, block_n, d_out), lambda i: (0, i, 0)),
            pl.BlockSpec((block_n, d_out), lambda i: (i, 0)),
            pl.BlockSpec((NC, block_n, LANES), lambda i: (0, i, 0)),
            pl.BlockSpec((1, d_out), lambda i: (0, 0)),
        ],
        out_specs=pl.BlockSpec((block_n, d_out), lambda i: (i, 0)),
        out_shape=jax.ShapeDtypeStruct((n, d_out), jnp.float32),
    )(accp, g, degp, b2d)


@jax.jit
def kernel(x, edge_index, edge_weight, W, b):
    n_nodes, _ = x.shape
    d_out = W.shape[1]
    n_edges = edge_index.shape[1]
    src = edge_index[0].astype(jnp.int32)
    dst = edge_index[1].astype(jnp.int32)
    ew = edge_weight.astype(jnp.float32)

    grp = NW * CHK
    tpc = -(-n_edges // grp)
    tpc += tpc % 2
    pad = tpc * grp - n_edges
    spread = jnp.arange(pad, dtype=jnp.int32) % n_nodes
    src = jnp.concatenate([src, spread])
    dst = jnp.concatenate([dst, spread])
    ew = jnp.concatenate([ew, jnp.zeros((pad,), jnp.float32)])
    src3 = src.reshape(NW, tpc, CHK)
    dst3 = dst.reshape(NW, tpc, CHK)
    ew3 = ew.reshape(NW, tpc, CHK)

    zero16 = jnp.zeros((n_nodes, LANES), jnp.float32)
    zero_d = jnp.zeros((n_nodes, d_out), jnp.float32)

    degp = _sc_deg(dst3, ew3, zero16, n_nodes, tpc)
    g = _tc_g(x, W, degp, block_n=2000)
    accp = _sc_msgs(src3, dst3, ew3, g, zero_d, n_nodes, d_out, tpc)
    return _tc_out(accp, g, degp, jnp.reshape(b, (1, d_out)), block_n=2000)


# speedup vs baseline: 2.8527x; 1.0080x over previous
import dataclasses
import functools
import jax
import jax.numpy as jnp
from jax import lax
from jax.experimental import pallas as pl
from jax.experimental.pallas import tpu as pltpu
from jax.experimental.pallas import tpu_sc as plsc

NC, NS, LANES = 2, 16, 16
NW = NC * NS
CHK = 128
RC = 400

_MESH = dict(core_axis_name="c", subcore_axis_name="s", num_cores=NC,
             num_subcores=NS)

_SC_PARAMS = dataclasses.replace(pltpu.CompilerParams(),
                                 needs_layout_passes=False,
                                 use_tc_tiling_on_sc=False)


def _sc_deg(dst3, ew3, zero16, n_nodes, tpc):
    @functools.partial(
        pl.kernel,
        out_type=jax.ShapeDtypeStruct((NC, n_nodes, LANES), jnp.float32),
        mesh=plsc.VectorSubcoreMesh(**_MESH),
        compiler_params=_SC_PARAMS,
        scratch_types=[
            pltpu.VMEM_SHARED((n_nodes, LANES), jnp.float32),
            pltpu.VMEM((tpc, CHK), jnp.int32),
            pltpu.VMEM((tpc, CHK), jnp.float32),
            pltpu.VMEM((CHK, LANES), jnp.float32),
            pltpu.VMEM((CHK, LANES), jnp.float32),
            pltpu.SemaphoreType.DMA,
            pltpu.SemaphoreType.DMA,
            pltpu.SemaphoreType.DMA,
        ],
    )
    def deg_kernel(dst3_hbm, ew3_hbm, zero_hbm, degp_hbm, shared, didx2, ewf,
                   rows0, rows1, sem, s0, s1):
        cid = lax.axis_index("c")
        sid = lax.axis_index("s")
        wid = sid * NC + cid
        n_rchunks = n_nodes // RC
        rows = (rows0, rows1)
        ssems = (s0, s1)

        cd = pltpu.async_copy(dst3_hbm.at[wid], didx2, s0)
        ce = pltpu.async_copy(ew3_hbm.at[wid], ewf, s1)

        @pl.loop(sid, n_rchunks, step=NS)
        def _(rc):
            r0 = pl.multiple_of(rc * RC, RC)
            pltpu.async_copy(zero_hbm.at[pl.ds(r0, RC)],
                             shared.at[pl.ds(r0, RC)], sem).wait()

        pltpu.async_copy(zero_hbm.at[pl.ds(0, CHK)], rows0, sem).wait()
        pltpu.async_copy(zero_hbm.at[pl.ds(0, CHK)], rows1, sem).wait()
        cd.wait()
        ce.wait()
        plsc.subcore_barrier()

        iota16 = lax.broadcasted_iota(jnp.int32, (LANES,), 0)
        lane0 = jnp.zeros((LANES,), jnp.int32)

        @pl.loop(0, tpc // 2)
        def _(k):
            for b in range(2):
                ci = k * 2 + b

                def wait_prev(b=b, ci=ci):
                    pltpu.make_async_copy(
                        rows[b], shared.at[didx2.at[ci]], ssems[b]).wait()

                pl.when(k >= 1)(wait_prev)

                for grp in range(CHK // LANES):
                    ew16 = ewf[ci, pl.ds(grp * LANES, LANES)]
                    plsc.store_scatter(rows[b], [iota16 + grp * LANES, lane0],
                                       ew16)

                pltpu.async_copy(rows[b], shared.at[didx2.at[ci]], ssems[b],
                                 add=True)

        pltpu.make_async_copy(rows0, shared.at[didx2.at[tpc - 2]], s0).wait()
        pltpu.make_async_copy(rows1, shared.at[didx2.at[tpc - 1]], s1).wait()
        plsc.subcore_barrier()

        @pl.loop(sid, n_rchunks, step=NS)
        def _(rc):
            r0 = pl.multiple_of(rc * RC, RC)
            pltpu.async_copy(shared.at[pl.ds(r0, RC)],
                             degp_hbm.at[cid, pl.ds(r0, RC)], sem).wait()

    return deg_kernel(dst3, ew3, zero16)


def _sc_msgs(src3, dst3, ew3, gbf, zero_d, n_nodes, d_out, tpc):
    n_vec32 = d_out // 32

    @functools.partial(
        pl.kernel,
        out_type=jax.ShapeDtypeStruct((NC, n_nodes, d_out), jnp.float32),
        mesh=plsc.VectorSubcoreMesh(**_MESH),
        compiler_params=_SC_PARAMS,
        scratch_types=[
            pltpu.VMEM_SHARED((n_nodes, d_out), jnp.float32),
            pltpu.VMEM((CHK,), jnp.int32),
            pltpu.VMEM((CHK,), jnp.int32),
            pltpu.VMEM((CHK,), jnp.int32),
            pltpu.VMEM((CHK,), jnp.int32),
            pltpu.VMEM((CHK,), jnp.float32),
            pltpu.VMEM((CHK,), jnp.float32),
            pltpu.VMEM((CHK, d_out), jnp.bfloat16),
            pltpu.VMEM((CHK, d_out), jnp.bfloat16),
            pltpu.VMEM((CHK, d_out), jnp.float32),
            pltpu.VMEM((CHK, d_out), jnp.float32),
            pltpu.SemaphoreType.DMA,
            pltpu.SemaphoreType.DMA,
            pltpu.SemaphoreType.DMA,
            pltpu.SemaphoreType.DMA,
            pltpu.SemaphoreType.DMA,
            pltpu.SemaphoreType.DMA,
            pltpu.SemaphoreType.DMA,
            pltpu.SemaphoreType.DMA,
            pltpu.SemaphoreType.DMA,
        ],
    )
    def msg_kernel(src3_hbm, dst3_hbm, ew3_hbm, gbf_hbm, zero_hbm, accp_hbm,
                   acc_sh, sidx0, sidx1, didx0, didx1, ewb0, ewb1,
                   rbf0, rbf1, rows0, rows1,
                   zsem, g0, g1, t0, t1, l0, l1, d0, d1):
        cid = lax.axis_index("c")
        sid = lax.axis_index("s")
        wid = sid * NC + cid
        n_rchunks = n_nodes // RC
        rows = (rows0, rows1)
        rbf = (rbf0, rbf1)
        sidx = (sidx0, sidx1)
        didx = (didx0, didx1)
        ewb = (ewb0, ewb1)
        gs = (g0, g1)
        ts = (t0, t1)
        ls = (l0, l1)
        ds_ = (d0, d1)

        def issue_loads(ci, b):
            pltpu.async_copy(src3_hbm.at[wid, ci], sidx[b], ls[b])
            pltpu.async_copy(ew3_hbm.at[wid, ci], ewb[b], ls[b])

        def wait_loads(b):
            pltpu.make_async_copy(src3_hbm.at[wid, 0], sidx[b], ls[b]).wait()
            pltpu.make_async_copy(ew3_hbm.at[wid, 0], ewb[b], ls[b]).wait()

        def issue_didx(ci, b):
            pltpu.async_copy(dst3_hbm.at[wid, ci], didx[b], ds_[b])

        def wait_didx(b):
            pltpu.make_async_copy(dst3_hbm.at[wid, 0], didx[b], ds_[b]).wait()

        @pl.loop(sid, n_rchunks, step=NS)
        def _(rc):
            r0 = pl.multiple_of(rc * RC, RC)
            pltpu.async_copy(zero_hbm.at[pl.ds(r0, RC)],
                             acc_sh.at[pl.ds(r0, RC)], zsem).wait()

        issue_loads(0, 0)
        issue_loads(1, 1)
        issue_didx(0, 0)
        wait_loads(0)
        pltpu.async_copy(gbf_hbm.at[sidx0], rbf0, g0)
        plsc.subcore_barrier()

        iota16 = lax.broadcasted_iota(jnp.int32, (LANES,), 0)

        @pl.loop(0, tpc // 2)
        def _(k):
            for b in range(2):
                ci = k * 2 + b

                pltpu.make_async_copy(gbf_hbm.at[sidx[b]], rbf[b],
                                      gs[b]).wait()

                def wait_scatter(b=b, ci=ci):
                    pltpu.make_async_copy(
                        rows[1 - b], acc_sh.at[didx[1 - b]], ts[1 - b]).wait()

                if b == 0:
                    pl.when(k >= 1)(wait_scatter)
                else:
                    wait_scatter()

                def issue_next_didx(b=b, ci=ci):
                    issue_didx(ci + 1, 1 - b)

                if b == 0:
                    issue_next_didx()
                else:
                    pl.when(k + 1 < tpc // 2)(issue_next_didx)

                def wait_idx(b=b):
                    wait_loads(1 - b)

                def issue_gather(b=b, ci=ci):
                    pltpu.async_copy(gbf_hbm.at[sidx[1 - b]], rbf[1 - b],
                                     gs[1 - b])

                if b == 0:
                    wait_idx()
                    issue_gather()
                else:
                    def prep_next(b=b, ci=ci):
                        wait_idx(b)
                        issue_gather(b, ci)

                    pl.when(k + 1 < tpc // 2)(prep_next)

                @plsc.parallel_loop(0, CHK, unroll=4)
                def _(e, b=b):
                    sp = plsc.load_gather(ewb[b],
                                          [lax.broadcast(e, (LANES,))])
                    esp = lax.broadcast(e, (LANES,))
                    for j in range(n_vec32):
                        y = rbf[b][e, pl.ds(j * 32, 32)]
                        ya, yb = plsc.unpack(y,
                                             format=plsc.PackFormat.INTERLEAVED)
                        plsc.store_scatter(
                            rows[b], [esp, j * 32 + 2 * iota16], ya * sp)
                        plsc.store_scatter(
                            rows[b], [esp, j * 32 + 2 * iota16 + 1], yb * sp)

                wait_didx(b)
                pltpu.async_copy(rows[b], acc_sh.at[didx[b]], ts[b],
                                 add=True)

                def issue_next_loads(b=b, ci=ci):
                    issue_loads(ci + 2, b)

                pl.when(ci + 2 < tpc)(issue_next_loads)

        pltpu.make_async_copy(rows1, acc_sh.at[didx1], t1).wait()
        plsc.subcore_barrier()

        @pl.loop(sid, n_rchunks, step=NS)
        def _(rc):
            r0 = pl.multiple_of(rc * RC, RC)
            pltpu.async_copy(acc_sh.at[pl.ds(r0, RC)],
                             accp_hbm.at[cid, pl.ds(r0, RC)], zsem).wait()

    return msg_kernel(src3, dst3, ew3, gbf, zero_d)


def _tc_g(x, W, degp, block_n):
    n, d_in = x.shape
    d_out = W.shape[1]

    def body(x_ref, w_ref, degp_ref, g_ref, gbf_ref):
        h = jnp.dot(x_ref[...], w_ref[...], preferred_element_type=jnp.float32)
        deg = degp_ref[0] + degp_ref[1] + 1.0
        dis = jnp.where(deg > 0, lax.rsqrt(jnp.maximum(deg, 1e-38)), 0.0)
        g = h * dis[:, 0:1]
        g_ref[...] = g
        gbf_ref[...] = g.astype(jnp.bfloat16)

    return pl.pallas_call(
        body,
        grid=(n // block_n,),
        in_specs=[
            pl.BlockSpec((block_n, d_in), lambda i: (i, 0)),
            pl.BlockSpec((d_in, d_out), lambda i: (0, 0)),
            pl.BlockSpec((NC, block_n, LANES), lambda i: (0, i, 0)),
        ],
        out_specs=[
            pl.BlockSpec((block_n, d_out), lambda i: (i, 0)),
            pl.BlockSpec((block_n, d_out), lambda i: (i, 0)),
        ],
        out_shape=[
            jax.ShapeDtypeStruct((n, d_out), jnp.float32),
            jax.ShapeDtypeStruct((n, d_out), jnp.bfloat16),
        ],
    )(x, W, degp)


def _tc_out(accp, g, degp, b2d, block_n):
    n, d_out = g.shape

    def body(accp_ref, g_ref, degp_ref, b_ref, o_ref):
        s = accp_ref[0] + accp_ref[1] + g_ref[...]
        deg = degp_ref[0] + degp_ref[1] + 1.0
        dis = jnp.where(deg > 0, lax.rsqrt(jnp.maximum(deg, 1e-38)), 0.0)
        z = s * dis[:, 0:1] + b_ref[...]
        m = jnp.max(z, axis=-1, keepdims=True)
        lse = m + jnp.log(jnp.sum(jnp.exp(z - m), axis=-1, keepdims=True))
        o_ref[...] = z - lse

    return pl.pallas_call(
        body,
        grid=(n // block_n,),
        in_specs=[
            pl.BlockSpec((NC, block_n, d_out), lambda i: (0, i, 0)),
            pl.BlockSpec((block_n, d_out), lambda i: (i, 0)),
            pl.BlockSpec((NC, block_n, LANES), lambda i: (0, i, 0)),
            pl.BlockSpec((1, d_out), lambda i: (0, 0)),
        ],
        out_specs=pl.BlockSpec((block_n, d_out), lambda i: (i, 0)),
        out_shape=jax.ShapeDtypeStruct((n, d_out), jnp.float32),
    )(accp, g, degp, b2d)


@jax.jit
def kernel(x, edge_index, edge_weight, W, b):
    n_nodes, _ = x.shape
    d_out = W.shape[1]
    n_edges = edge_index.shape[1]
    src = edge_index[0].astype(jnp.int32)
    dst = edge_index[1].astype(jnp.int32)
    ew = edge_weight.astype(jnp.float32)

    grp = NW * CHK
    tpc = -(-n_edges // grp)
    tpc += tpc % 2
    pad = tpc * grp - n_edges
    spread = jnp.arange(pad, dtype=jnp.int32) % n_nodes
    src = jnp.concatenate([src, spread])
    dst = jnp.concatenate([dst, spread])
    ew = jnp.concatenate([ew, jnp.zeros((pad,), jnp.float32)])
    src3 = src.reshape(NW, tpc, CHK)
    dst3 = dst.reshape(NW, tpc, CHK)
    ew3 = ew.reshape(NW, tpc, CHK)

    zero16 = jnp.zeros((n_nodes, LANES), jnp.float32)
    zero_d = jnp.zeros((n_nodes, d_out), jnp.float32)

    degp = _sc_deg(dst3, ew3, zero16, n_nodes, tpc)
    g, gbf = _tc_g(x, W, degp, block_n=2000)
    accp = _sc_msgs(src3, dst3, ew3, gbf, zero_d, n_nodes, d_out, tpc)
    return _tc_out(accp, g, degp, jnp.reshape(b, (1, d_out)), block_n=2000)
